# baseline (speedup 1.0000x reference)
import functools

import jax
import jax.numpy as jnp
from jax import lax
from jax.experimental import pallas as pl
from jax.experimental.pallas import tpu as pltpu
import jax.experimental.pallas.tpu_sc as plsc

NUM_NODES = 100000
DIM_PE = 64
DIM_IN = 128
DIM_H = 192

POS_WALKS = 1408
NEG_WALKS = 7040
ALL_WALKS = POS_WALKS + NEG_WALKS
WL = 10
PPW = WL - 1
POS_PAIRS = POS_WALKS * PPW
ALL_PAIRS = ALL_WALKS * PPW

NC, NS, L = 2, 16, 16
NW = NC * NS

WPS = ALL_WALKS // NW
CW = 12
NCH = WPS // CW
KPE = DIM_PE // L


RC = 200
RK = 15
RTAIL = NUM_NODES // RC - RK * NW


def _relay_body(emb_hbm, out_hbm, vin0, vin1, vout0, vout1,
                si0, si1, so0, so1):
    wid = lax.axis_index("s") * NC + lax.axis_index("c")
    vins = [vin0, vin1]
    vouts = [vout0, vout1]
    sins = [si0, si1]
    souts = [so0, so1]

    def start_in(k):
        a = (k * NW + wid) * RC
        return pltpu.async_copy(emb_hbm.at[pl.ds(a, RC), :],
                                vins[k & 1], sins[k & 1])

    def compute(b):
        vi = vins[b]
        vo = vouts[b]

        def rows4(i, _):
            r0 = i * 4
            for dr in range(4):
                for kk in range(KPE):
                    vo[r0 + dr, pl.ds(kk * L, L)] = \
                        vi[r0 + dr, pl.ds(kk * L, L)]
            return 0

        lax.fori_loop(0, RC // 4, rows4, 0)

    def start_out(k):
        a = (k * NW + wid) * RC
        return pltpu.async_copy(vouts[k & 1],
                                out_hbm.at[pl.ds(a, RC), :], souts[k & 1])

    ins = {0: start_in(0)}
    outs = {}
    for k in range(RK):
        if k + 1 < RK:
            ins[k + 1] = start_in(k + 1)
        ins.pop(k).wait()
        if k - 2 >= 0:
            outs.pop(k - 2).wait()
        compute(k & 1)
        outs[k] = start_out(k)
    for k in (RK - 2, RK - 1):
        outs.pop(k).wait()

    @pl.when(wid < RTAIL)
    def _():
        a = (RK * NW + wid) * RC
        pltpu.sync_copy(emb_hbm.at[pl.ds(a, RC), :], vins[0])
        compute(0)
        pltpu.sync_copy(vouts[0], out_hbm.at[pl.ds(a, RC), :])


@functools.cache
def _get_relay_call():
    return functools.partial(
        pl.kernel,
        out_type=jax.ShapeDtypeStruct((NUM_NODES, 128), jnp.float32),
        mesh=plsc.VectorSubcoreMesh(core_axis_name="c", subcore_axis_name="s",
                                    num_cores=NC, num_subcores=NS),
        scratch_types=[
            pltpu.VMEM((RC, DIM_PE), jnp.float32),
            pltpu.VMEM((RC, DIM_PE), jnp.float32),
            pltpu.VMEM((RC, 128), jnp.float32),
            pltpu.VMEM((RC, 128), jnp.float32),
            pltpu.SemaphoreType.DMA,
            pltpu.SemaphoreType.DMA,
            pltpu.SemaphoreType.DMA,
            pltpu.SemaphoreType.DMA,
        ],
    )(_relay_body)


def _sc_body(emb_hbm, idx_hbm, out_hbm,
             idx_all, rows0, rows1, out0, out1, gs0, gs1, os0, os1):
    wid = lax.axis_index("s") * NC + lax.axis_index("c")
    base_w = wid * WPS
    pltpu.sync_copy(idx_hbm.at[pl.ds(base_w * WL, WPS * WL)], idx_all)

    rows = [rows0, rows1]
    outs = [out0, out1]
    gsem = [gs0, gs1]
    osem = [os0, os1]

    def start_gather(c):
        return pltpu.async_copy(
            emb_hbm.at[idx_all.at[pl.ds(c * (CW * WL), CW * WL)]],
            rows[c & 1], gsem[c & 1])

    gathers = {0: start_gather(0)}
    outcps = {}
    for c in range(NCH):
        cb = c & 1
        if c + 1 < NCH:
            gathers[c + 1] = start_gather(c + 1)
        gathers.pop(c).wait()
        if c - 2 >= 0:
            outcps.pop(c - 2).wait()
        rv = rows[cb]
        ov = outs[cb]

        def walk_body(w, _, rv=rv, ov=ov):
            r0 = w * WL
            s_regs = [rv[r0, pl.ds(k * L, L)] for k in range(KPE)]
            for j in range(1, WL):
                acc = s_regs[0] * rv[r0 + j, pl.ds(0, L)]
                for k in range(1, KPE):
                    acc = acc + s_regs[k] * rv[r0 + j, pl.ds(k * L, L)]
                ov[pl.ds((w * PPW + j - 1) * L, L)] = acc
            return 0

        lax.fori_loop(0, CW, walk_body, 0)
        outcps[c] = pltpu.async_copy(
            ov,
            out_hbm.at[pl.ds((base_w + c * CW) * PPW * L, CW * PPW * L)],
            osem[cb])
    for c in (NCH - 2, NCH - 1):
        outcps.pop(c).wait()


@functools.cache
def _get_sc_call():
    return functools.partial(
        pl.kernel,
        out_type=jax.ShapeDtypeStruct((ALL_PAIRS * L,), jnp.float32),
        mesh=plsc.VectorSubcoreMesh(core_axis_name="c", subcore_axis_name="s",
                                    num_cores=NC, num_subcores=NS),
        scratch_types=[
            pltpu.VMEM((WPS * WL,), jnp.int32),
            pltpu.VMEM((CW * WL, 128), jnp.float32),
            pltpu.VMEM((CW * WL, 128), jnp.float32),
            pltpu.VMEM((CW * PPW * L,), jnp.float32),
            pltpu.VMEM((CW * PPW * L,), jnp.float32),
            pltpu.SemaphoreType.DMA,
            pltpu.SemaphoreType.DMA,
            pltpu.SemaphoreType.DMA,
            pltpu.SemaphoreType.DMA,
        ],
    )(_sc_body)


def _dense_body(x_ref, emb_ref, lxw_ref, encw_ref, lxb_ref, encb_ref, out_ref):
    h = lax.dot_general(x_ref[...], lxw_ref[...],
                        (((1,), (1,)), ((), ())),
                        preferred_element_type=jnp.float32)
    out_ref[:, :DIM_H] = h + lxb_ref[...]
    e = emb_ref[...]
    nrm = jnp.sqrt(jnp.sum(e * e, axis=1, keepdims=True))
    e = e / jnp.maximum(nrm, 1e-12)
    pe = lax.dot_general(e, encw_ref[...],
                         (((1,), (1,)), ((), ())),
                         preferred_element_type=jnp.float32)
    out_ref[:, DIM_H:] = pe + encb_ref[...]


def _dense_call(x, emb, lx_W, enc_W, lx_b2, enc_b2, rows_per_block=10000):
    n = x.shape[0]
    grid = (n // rows_per_block,)
    return pl.pallas_call(
        _dense_body,
        grid=grid,
        in_specs=[
            pl.BlockSpec((rows_per_block, DIM_IN), lambda i: (i, 0)),
            pl.BlockSpec((rows_per_block, DIM_PE), lambda i: (i, 0)),
            pl.BlockSpec((DIM_H, DIM_IN), lambda i: (0, 0)),
            pl.BlockSpec((DIM_PE, DIM_PE), lambda i: (0, 0)),
            pl.BlockSpec((1, DIM_H), lambda i: (0, 0)),
            pl.BlockSpec((1, DIM_PE), lambda i: (0, 0)),
        ],
        out_specs=pl.BlockSpec((rows_per_block, DIM_H + DIM_PE),
                               lambda i: (i, 0)),
        out_shape=jax.ShapeDtypeStruct((n, DIM_H + DIM_PE), jnp.float32),
    )(x, emb, lx_W, enc_W, lx_b2, enc_b2)


def _stable_log_sigmoid(z):
    return jnp.minimum(z, 0.0) - jnp.log(1.0 + jnp.exp(-jnp.abs(z)))


GPR = 128 // L
S_ROWS = ALL_PAIRS * L // 128
POS_ROWS = POS_PAIRS * L // 128


def _loss_body(s_ref, out_ref):
    lane = lax.broadcasted_iota(jnp.int32, (128, GPR), 0)
    grp = lax.broadcasted_iota(jnp.int32, (128, GPR), 1)
    m = (lane // L == grp).astype(jnp.float32)
    z = lax.dot_general(s_ref[...], m, (((1,), (0,)), ((), ())),
                        preferred_element_type=jnp.float32)
    lp = jnp.sum(_stable_log_sigmoid(z[:POS_ROWS, :])) / POS_PAIRS
    ln = jnp.sum(_stable_log_sigmoid(-z[POS_ROWS:, :])) / (ALL_PAIRS - POS_PAIRS)
    out_ref[...] = jnp.full((1, 1), -(lp + ln), dtype=jnp.float32)


def _loss_call(s):
    return pl.pallas_call(
        _loss_body,
        out_shape=jax.ShapeDtypeStruct((1, 1), jnp.float32),
    )(s.reshape(S_ROWS, 128))


def kernel(x, pos_rw, neg_rw, emb, enc_W, enc_b, lx_W, lx_b):
    all_idx = jnp.concatenate([pos_rw.astype(jnp.int32).reshape(-1),
                               neg_rw.astype(jnp.int32).reshape(-1)])
    emb128 = _get_relay_call()(emb)
    s = _get_sc_call()(emb128, all_idx)
    out = _dense_call(x, emb, lx_W, enc_W,
                      lx_b.reshape(1, DIM_H), enc_b.reshape(1, DIM_PE))
    loss = _loss_call(s)
    return out, loss[0, 0]
